# Initial kernel scaffold; baseline (speedup 1.0000x reference)
import jax
import jax.numpy as jnp
from jax.experimental import pallas as pl


def kernel(local_energies, atomic_numbers, per_element_scaling):
    raise NotImplementedError("write your pallas kernel here")


# speedup vs baseline: 24.1040x; 24.1040x over previous
import functools

import jax
import jax.numpy as jnp
from jax import lax
from jax.experimental import pallas as pl
from jax.experimental.pallas import tpu as pltpu
from jax.experimental.pallas import tpu_sc as plsc

_L = 16


@functools.lru_cache(maxsize=None)
def _build(n: int, num_elements: int):
    info = plsc.get_sparse_core_info()
    nc, ns = info.num_cores, info.num_subcores
    nw = nc * ns

    chunk = ((n + nw - 1) // nw + _L - 1) // _L * _L
    tail = n - (nw - 1) * chunk
    assert tail > 0 and tail % _L == 0 and n % _L == 0 and chunk % 8 == 0

    mesh = plsc.VectorSubcoreMesh(core_axis_name="c", subcore_axis_name="s")

    @functools.partial(
        pl.kernel,
        mesh=mesh,
        compiler_params=pltpu.CompilerParams(needs_layout_passes=False),
        out_type=jax.ShapeDtypeStruct((n,), jnp.float32),
        scratch_types=[
            pltpu.VMEM((num_elements,), jnp.float32),
            pltpu.VMEM((chunk,), jnp.int32),
            pltpu.VMEM((chunk,), jnp.float32),
            pltpu.VMEM((chunk,), jnp.float32),
        ],
    )
    def sc_kernel(le_hbm, an_hbm, tab_hbm, out_hbm, tab_v, idx_v, le_v, out_v):
        wid = lax.axis_index("s") * nc + lax.axis_index("c")
        pltpu.sync_copy(tab_hbm, tab_v)

        def do_chunk(base, count):
            pltpu.sync_copy(an_hbm.at[pl.ds(base, count)],
                            idx_v.at[pl.ds(0, count)])
            pltpu.sync_copy(le_hbm.at[pl.ds(base, count)],
                            le_v.at[pl.ds(0, count)])

            def body(j, carry):
                s = pl.ds(j * _L, _L)
                scales = plsc.load_gather(tab_v, [idx_v[s]])
                out_v[s] = scales * le_v[s]
                return carry

            lax.fori_loop(0, count // _L, body, 0)
            pltpu.sync_copy(out_v.at[pl.ds(0, count)],
                            out_hbm.at[pl.ds(base, count)])

        @pl.when(wid < nw - 1)
        def _():
            do_chunk(wid * chunk, chunk)

        @pl.when(wid == nw - 1)
        def _():
            do_chunk((nw - 1) * chunk, tail)

    return sc_kernel


def kernel(local_energies, atomic_numbers, per_element_scaling):
    n = local_energies.shape[0]
    table = per_element_scaling.reshape(-1).astype(jnp.float32)
    idx = atomic_numbers.astype(jnp.int32)
    fn = _build(n, table.shape[0])
    return fn(local_energies.astype(jnp.float32), idx, table)


# speedup vs baseline: 24.6487x; 1.0226x over previous
import functools

import jax
import jax.numpy as jnp
from jax import lax
from jax.experimental import pallas as pl
from jax.experimental.pallas import tpu as pltpu
from jax.experimental.pallas import tpu_sc as plsc

_L = 16


@functools.lru_cache(maxsize=None)
def _build(n: int, num_elements: int):
    info = plsc.get_sparse_core_info()
    nc, ns = info.num_cores, info.num_subcores
    nw = nc * ns

    chunk = ((n + nw - 1) // nw + _L - 1) // _L * _L
    assert n >= chunk and n % 8 == 0 and chunk % _L == 0

    mesh = plsc.VectorSubcoreMesh(core_axis_name="c", subcore_axis_name="s")

    @functools.partial(
        pl.kernel,
        mesh=mesh,
        compiler_params=pltpu.CompilerParams(needs_layout_passes=False),
        out_type=jax.ShapeDtypeStruct((n,), jnp.float32),
        scratch_types=[
            pltpu.VMEM((num_elements,), jnp.float32),
            pltpu.VMEM((chunk,), jnp.int32),
            pltpu.VMEM((chunk,), jnp.float32),
            pltpu.VMEM((chunk,), jnp.float32),
            pltpu.SemaphoreType.DMA,
        ],
    )
    def sc_kernel(le_hbm, an_hbm, tab_hbm, out_hbm,
                  tab_v, idx_v, le_v, out_v, sem):
        wid = lax.axis_index("s") * nc + lax.axis_index("c")
        base = jnp.minimum(wid * chunk, n - chunk)

        c1 = pltpu.async_copy(tab_hbm, tab_v, sem)
        c2 = pltpu.async_copy(an_hbm.at[pl.ds(base, chunk)], idx_v, sem)
        c3 = pltpu.async_copy(le_hbm.at[pl.ds(base, chunk)], le_v, sem)
        c1.wait()
        c2.wait()
        c3.wait()

        def body(j, carry):
            s = pl.ds(j * _L, _L)
            out_v[s] = plsc.load_gather(tab_v, [idx_v[s]]) * le_v[s]
            return carry

        lax.fori_loop(0, chunk // _L, body, 0, unroll=8)
        pltpu.sync_copy(out_v, out_hbm.at[pl.ds(base, chunk)])

    return sc_kernel


def kernel(local_energies, atomic_numbers, per_element_scaling):
    n = local_energies.shape[0]
    table = per_element_scaling.reshape(-1).astype(jnp.float32)
    idx = atomic_numbers.astype(jnp.int32)
    fn = _build(n, table.shape[0])
    return fn(local_energies.astype(jnp.float32), idx, table)


# speedup vs baseline: 25.6861x; 1.0421x over previous
import functools

import jax
import jax.numpy as jnp
from jax import lax
from jax.experimental import pallas as pl
from jax.experimental.pallas import tpu as pltpu
from jax.experimental.pallas import tpu_sc as plsc

_L = 16
_NQ = 4


@functools.lru_cache(maxsize=None)
def _build(n: int, num_elements: int):
    info = plsc.get_sparse_core_info()
    nc, ns = info.num_cores, info.num_subcores
    nw = nc * ns

    q = _L * _NQ
    chunk = ((n + nw - 1) // nw + q - 1) // q * q
    quarter = chunk // _NQ
    assert n >= chunk and n % 8 == 0 and quarter % _L == 0

    mesh = plsc.VectorSubcoreMesh(core_axis_name="c", subcore_axis_name="s")

    @functools.partial(
        pl.kernel,
        mesh=mesh,
        compiler_params=pltpu.CompilerParams(needs_layout_passes=False),
        out_type=jax.ShapeDtypeStruct((n,), jnp.float32),
        scratch_types=[
            pltpu.VMEM((num_elements,), jnp.float32),
            pltpu.VMEM((chunk,), jnp.int32),
            pltpu.VMEM((chunk,), jnp.float32),
            pltpu.VMEM((chunk,), jnp.float32),
        ] + [pltpu.SemaphoreType.DMA] * _NQ
          + [pltpu.SemaphoreType.DMA],
    )
    def sc_kernel(le_hbm, an_hbm, tab_hbm, out_hbm,
                  tab_v, idx_v, le_v, out_v, *sems):
        in_sems, out_sem = sems[:_NQ], sems[_NQ]
        wid = lax.axis_index("s") * nc + lax.axis_index("c")
        base = jnp.minimum(wid * chunk, n - chunk)

        ct = pltpu.async_copy(tab_hbm, tab_v, in_sems[0])
        copies = []
        for k in range(_NQ):
            o = k * quarter
            copies.append((
                pltpu.async_copy(an_hbm.at[pl.ds(base + o, quarter)],
                                 idx_v.at[pl.ds(o, quarter)], in_sems[k]),
                pltpu.async_copy(le_hbm.at[pl.ds(base + o, quarter)],
                                 le_v.at[pl.ds(o, quarter)], in_sems[k]),
            ))

        outs = []
        for k in range(_NQ):
            o = k * quarter
            if k == 0:
                ct.wait()
            ci, cl = copies[k]
            ci.wait()
            cl.wait()

            @plsc.parallel_loop(o, o + quarter, step=_L, unroll=8)
            def _(i):
                s = pl.ds(i, _L)
                out_v[s] = plsc.load_gather(tab_v, [idx_v[s]]) * le_v[s]

            outs.append(pltpu.async_copy(
                out_v.at[pl.ds(o, quarter)],
                out_hbm.at[pl.ds(base + o, quarter)], out_sem))

        for c in outs:
            c.wait()

    return sc_kernel


def kernel(local_energies, atomic_numbers, per_element_scaling):
    n = local_energies.shape[0]
    table = per_element_scaling.reshape(-1).astype(jnp.float32)
    idx = atomic_numbers.astype(jnp.int32)
    fn = _build(n, table.shape[0])
    return fn(local_energies.astype(jnp.float32), idx, table)
